# Initial kernel scaffold; baseline (speedup 1.0000x reference)
import jax
import jax.numpy as jnp
from jax.experimental import pallas as pl
from jax.experimental.pallas import tpu as pltpu

B = 1
S = 2048
D = 768
N = 4096
K = 32
TBLK = 256
GRID = S // TBLK


def _gate_block(scores, tau, k):
    raw = scores - tau
    gate = jnp.where(raw > 0, raw, 1e-08 * jnp.exp(raw))
    eg = jnp.exp(gate) - 1.0
    bits = jax.lax.bitcast_convert_type(eg, jnp.int32)
    t = scores.shape[0]
    lo0 = jnp.zeros((t, 1), jnp.int32)
    hi0 = jnp.full((t, 1), 0x7F800001, jnp.int32)

    def body(_, carry):
        lo, hi = carry
        mid = lo + ((hi - lo) >> 1)
        cnt = jnp.sum((bits >= mid).astype(jnp.float32), axis=1, keepdims=True)
        ok = cnt >= float(k)
        return jnp.where(ok, mid, lo), jnp.where(ok, mid, hi)

    lo, _ = jax.lax.fori_loop(0, 31, body, (lo0, hi0))
    thr = jax.lax.bitcast_convert_type(lo, jnp.float32)
    eg = jnp.where(eg >= thr, eg, 0.0)
    s = jnp.sum(eg, axis=1, keepdims=True) + 1e-08
    m = jnp.max(eg, axis=1, keepdims=True)
    return eg / s * jnp.tanh(m)


def _aux_of(sum_ref):
    mean = sum_ref[...] / float(B * S)
    return jnp.sum((mean - 1.0 / N) ** 2) * float(N)


def _qk_kernel(x_ref, emb_ref, w_ref, b_ref, gq_ref, gk_ref, aux_ref,
               sq_ref, sk_ref):
    i = pl.program_id(0)

    @pl.when(i == 0)
    def _():
        sq_ref[...] = jnp.zeros_like(sq_ref)
        sk_ref[...] = jnp.zeros_like(sk_ref)

    x = x_ref[...]
    scores = jax.lax.dot_general(
        x, emb_ref[...], dimension_numbers=(((1,), (1,)), ((), ())),
        preferred_element_type=jnp.float32)
    tau = jnp.dot(x, w_ref[...], preferred_element_type=jnp.float32) + b_ref[...]
    gq = _gate_block(scores, tau[:, 0:1], K)
    gk = _gate_block(scores, tau[:, 1:2], K)
    gq_ref[...] = gq
    gk_ref[...] = gk
    sq_ref[...] += jnp.sum(gq, axis=0, keepdims=True)
    sk_ref[...] += jnp.sum(gk, axis=0, keepdims=True)
    aux_ref[0, 0] = _aux_of(sq_ref) + _aux_of(sk_ref)


def _single_kernel(x_ref, emb_ref, w_ref, b_ref, auxin_ref, g_ref, aux_ref,
                   s_ref):
    i = pl.program_id(0)

    @pl.when(i == 0)
    def _():
        s_ref[...] = jnp.zeros_like(s_ref)

    x = x_ref[...]
    scores = jax.lax.dot_general(
        x, emb_ref[...], dimension_numbers=(((1,), (1,)), ((), ())),
        preferred_element_type=jnp.float32)
    tau = jnp.dot(x, w_ref[...], preferred_element_type=jnp.float32) + b_ref[...]
    g = _gate_block(scores, tau, K)
    g_ref[...] = g
    s_ref[...] += jnp.sum(g, axis=0, keepdims=True)
    aux_ref[0, 0] = auxin_ref[0, 0] + _aux_of(s_ref)


def _common_specs():
    x_spec = pl.BlockSpec((TBLK, D), lambda i: (i, 0))
    emb_spec = pl.BlockSpec((N, D), lambda i: (0, 0))
    g_spec = pl.BlockSpec((TBLK, N), lambda i: (i, 0))
    aux_spec = pl.BlockSpec((1, 1), lambda i: (0, 0))
    return x_spec, emb_spec, g_spec, aux_spec


def kernel(x, qk_emb, v_emb, know_emb, W_tau_attn, b_tau_attn, W_tau_know,
           b_tau_know):
    x2 = x.reshape(S, D)
    x_spec, emb_spec, g_spec, aux_spec = _common_specs()
    w3_spec = pl.BlockSpec((D, 3), lambda i: (0, 0))
    w1_spec = pl.BlockSpec((D, 1), lambda i: (0, 0))
    b3 = b_tau_attn.reshape(1, 3)
    sum_scratch = pltpu.VMEM((1, N), jnp.float32)

    gq, gk, aux_qk = pl.pallas_call(
        _qk_kernel,
        grid=(GRID,),
        in_specs=[x_spec, emb_spec, w3_spec, aux_spec],
        out_specs=[g_spec, g_spec, aux_spec],
        out_shape=[
            jax.ShapeDtypeStruct((S, N), jnp.float32),
            jax.ShapeDtypeStruct((S, N), jnp.float32),
            jax.ShapeDtypeStruct((1, 1), jnp.float32),
        ],
        scratch_shapes=[sum_scratch, sum_scratch],
    )(x2, qk_emb, W_tau_attn, b3)

    gv, aux_attn = pl.pallas_call(
        _single_kernel,
        grid=(GRID,),
        in_specs=[x_spec, emb_spec, w1_spec, aux_spec, aux_spec],
        out_specs=[g_spec, aux_spec],
        out_shape=[
            jax.ShapeDtypeStruct((S, N), jnp.float32),
            jax.ShapeDtypeStruct((1, 1), jnp.float32),
        ],
        scratch_shapes=[sum_scratch],
    )(x2, v_emb, W_tau_attn[:, 2:3], b_tau_attn[2].reshape(1, 1), aux_qk)

    gknow, aux_know = pl.pallas_call(
        _single_kernel,
        grid=(GRID,),
        in_specs=[x_spec, emb_spec, w1_spec, aux_spec, aux_spec],
        out_specs=[g_spec, aux_spec],
        out_shape=[
            jax.ShapeDtypeStruct((S, N), jnp.float32),
            jax.ShapeDtypeStruct((1, 1), jnp.float32),
        ],
        scratch_shapes=[sum_scratch],
    )(x2, know_emb, W_tau_know, b_tau_know.reshape(1, 1),
      jnp.zeros((1, 1), jnp.float32))

    shape = (B, S, N)
    return (gq.reshape(shape), gk.reshape(shape), gv.reshape(shape),
            aux_attn.reshape(()), gknow.reshape(shape), aux_know.reshape(()))


# speedup vs baseline: 11.0730x; 11.0730x over previous
import jax
import jax.numpy as jnp
from jax.experimental import pallas as pl
from jax.experimental.pallas import tpu as pltpu

B = 1
S = 2048
D = 768
N = 4096
K = 32
TBLK = 256
GRID = S // TBLK


def _gate_block(scores, tau, k):
    raw = scores - tau
    gate = jnp.where(raw > 0, raw, 1e-08 * jnp.exp(raw))
    eg = jnp.exp(gate) - 1.0
    bits = jax.lax.bitcast_convert_type(eg, jnp.int32)
    t = scores.shape[0]
    lo0 = jnp.zeros((t, 1), jnp.int32)
    hi0 = jnp.full((t, 1), 0x7F800001, jnp.int32)

    def body(_, carry):
        lo, hi = carry
        mid = lo + ((hi - lo) >> 1)
        cnt = jnp.sum((bits >= mid).astype(jnp.float32), axis=1, keepdims=True)
        ok = cnt >= float(k)
        return jnp.where(ok, mid, lo), jnp.where(ok, hi, mid)

    lo, _ = jax.lax.fori_loop(0, 31, body, (lo0, hi0))
    thr = jax.lax.bitcast_convert_type(lo, jnp.float32)
    eg = jnp.where(eg >= thr, eg, 0.0)
    s = jnp.sum(eg, axis=1, keepdims=True) + 1e-08
    m = jnp.max(eg, axis=1, keepdims=True)
    return eg / s * jnp.tanh(m)


def _aux_of(sum_ref):
    mean = sum_ref[...] / float(B * S)
    return jnp.sum((mean - 1.0 / N) ** 2) * float(N)


def _qk_kernel(x_ref, emb_ref, w_ref, b_ref, gq_ref, gk_ref, aux_ref,
               sq_ref, sk_ref):
    i = pl.program_id(0)

    @pl.when(i == 0)
    def _():
        sq_ref[...] = jnp.zeros_like(sq_ref)
        sk_ref[...] = jnp.zeros_like(sk_ref)

    x = x_ref[...]
    scores = jax.lax.dot_general(
        x / 0.9, emb_ref[...], dimension_numbers=(((1,), (1,)), ((), ())),
        preferred_element_type=jnp.float32)
    tau = jnp.dot(x, w_ref[...], preferred_element_type=jnp.float32) + b_ref[...]
    gq = _gate_block(scores, tau[:, 0:1], K)
    gk = _gate_block(scores, tau[:, 1:2], K)
    gq_ref[...] = gq
    gk_ref[...] = gk
    sq_ref[...] += jnp.sum(gq, axis=0, keepdims=True)
    sk_ref[...] += jnp.sum(gk, axis=0, keepdims=True)
    aux_ref[...] = jnp.reshape(_aux_of(sq_ref) + _aux_of(sk_ref), (1, 1))


def _single_kernel(x_ref, emb_ref, w_ref, b_ref, auxin_ref, g_ref, aux_ref,
                   s_ref):
    i = pl.program_id(0)

    @pl.when(i == 0)
    def _():
        s_ref[...] = jnp.zeros_like(s_ref)

    x = x_ref[...]
    scores = jax.lax.dot_general(
        x / 0.9, emb_ref[...], dimension_numbers=(((1,), (1,)), ((), ())),
        preferred_element_type=jnp.float32)
    tau = jnp.dot(x, w_ref[...], preferred_element_type=jnp.float32) + b_ref[...]
    g = _gate_block(scores, tau, K)
    g_ref[...] = g
    s_ref[...] += jnp.sum(g, axis=0, keepdims=True)
    aux_ref[...] = auxin_ref[...] + jnp.reshape(_aux_of(s_ref), (1, 1))


def _common_specs():
    x_spec = pl.BlockSpec((TBLK, D), lambda i: (i, 0))
    emb_spec = pl.BlockSpec((N, D), lambda i: (0, 0))
    g_spec = pl.BlockSpec((TBLK, N), lambda i: (i, 0))
    aux_spec = pl.BlockSpec((1, 1), lambda i: (0, 0))
    return x_spec, emb_spec, g_spec, aux_spec


def kernel(x, qk_emb, v_emb, know_emb, W_tau_attn, b_tau_attn, W_tau_know,
           b_tau_know):
    x2 = x.reshape(S, D)
    x_spec, emb_spec, g_spec, aux_spec = _common_specs()
    w3_spec = pl.BlockSpec((D, 3), lambda i: (0, 0))
    w1_spec = pl.BlockSpec((D, 1), lambda i: (0, 0))
    b3 = b_tau_attn.reshape(1, 3)
    sum_scratch = pltpu.VMEM((1, N), jnp.float32)

    gq, gk, aux_qk = pl.pallas_call(
        _qk_kernel,
        grid=(GRID,),
        in_specs=[x_spec, emb_spec, w3_spec, pl.BlockSpec((1, 3), lambda i: (0, 0))],
        out_specs=[g_spec, g_spec, aux_spec],
        out_shape=[
            jax.ShapeDtypeStruct((S, N), jnp.float32),
            jax.ShapeDtypeStruct((S, N), jnp.float32),
            jax.ShapeDtypeStruct((1, 1), jnp.float32),
        ],
        scratch_shapes=[sum_scratch, sum_scratch],
    )(x2, qk_emb, W_tau_attn, b3)

    gv, aux_attn = pl.pallas_call(
        _single_kernel,
        grid=(GRID,),
        in_specs=[x_spec, emb_spec, w1_spec, aux_spec, aux_spec],
        out_specs=[g_spec, aux_spec],
        out_shape=[
            jax.ShapeDtypeStruct((S, N), jnp.float32),
            jax.ShapeDtypeStruct((1, 1), jnp.float32),
        ],
        scratch_shapes=[sum_scratch],
    )(x2, v_emb, W_tau_attn[:, 2:3], b_tau_attn[2].reshape(1, 1), aux_qk)

    gknow, aux_know = pl.pallas_call(
        _single_kernel,
        grid=(GRID,),
        in_specs=[x_spec, emb_spec, w1_spec, aux_spec, aux_spec],
        out_specs=[g_spec, aux_spec],
        out_shape=[
            jax.ShapeDtypeStruct((S, N), jnp.float32),
            jax.ShapeDtypeStruct((1, 1), jnp.float32),
        ],
        scratch_shapes=[sum_scratch],
    )(x2, know_emb, W_tau_know, b_tau_know.reshape(1, 1),
      jnp.zeros((1, 1), jnp.float32))

    shape = (B, S, N)
    return (gq.reshape(shape), gk.reshape(shape), gv.reshape(shape),
            aux_attn.reshape(()), gknow.reshape(shape), aux_know.reshape(()))


# speedup vs baseline: 14.6910x; 1.3267x over previous
import jax
import jax.numpy as jnp
from jax.experimental import pallas as pl
from jax.experimental.pallas import tpu as pltpu

B = 1
S = 2048
D = 768
N = 4096
K = 32
TBLK = 256
GRID = S // TBLK


def _score_threshold(scores, k):
    b = jax.lax.bitcast_convert_type(scores, jnp.int32)
    skey = jnp.where(b < 0, b ^ 0x7FFFFFFF, b)
    gm = skey[:, 0:128]
    for c in range(1, N // 128):
        gm = jnp.maximum(gm, skey[:, c * 128:(c + 1) * 128])
    lo0 = jnp.min(gm, axis=1, keepdims=True)
    hi0 = jnp.max(gm, axis=1, keepdims=True) + 1

    def cond(carry):
        lo, hi = carry
        return jnp.any(hi - lo > 1)

    def body(carry):
        lo, hi = carry
        mid = (lo & hi) + ((lo ^ hi) >> 1)
        cnt = jnp.sum((skey >= mid).astype(jnp.float32), axis=1, keepdims=True)
        ok = cnt >= float(k)
        return jnp.where(ok, mid, lo), jnp.where(ok, hi, mid)

    lo, _ = jax.lax.while_loop(cond, body, (lo0, hi0))
    bthr = jnp.where(lo < 0, lo ^ 0x7FFFFFFF, lo)
    return jax.lax.bitcast_convert_type(bthr, jnp.float32)


def _apply_gate(scores, tau, thr):
    raw = scores - tau
    keep = (scores >= thr) & (raw > 0)
    eg = jnp.where(keep, jnp.exp(raw) - 1.0, 0.0)
    s = jnp.sum(eg, axis=1, keepdims=True) + 1e-08
    m = jnp.max(eg, axis=1, keepdims=True)
    return eg / s * jnp.tanh(m)


def _aux_of(sum_ref):
    mean = sum_ref[...] / float(B * S)
    return jnp.sum((mean - 1.0 / N) ** 2) * float(N)


def _qk_kernel(x_ref, emb_ref, w_ref, b_ref, gq_ref, gk_ref, aux_ref,
               sq_ref, sk_ref):
    i = pl.program_id(0)

    @pl.when(i == 0)
    def _():
        sq_ref[...] = jnp.zeros_like(sq_ref)
        sk_ref[...] = jnp.zeros_like(sk_ref)

    x = x_ref[...]
    scores = jax.lax.dot_general(
        x / 0.9, emb_ref[...], dimension_numbers=(((1,), (1,)), ((), ())),
        preferred_element_type=jnp.float32)
    tau = jnp.dot(x, w_ref[...], preferred_element_type=jnp.float32) + b_ref[...]
    thr = _score_threshold(scores, K)
    gq = _apply_gate(scores, tau[:, 0:1], thr)
    gk = _apply_gate(scores, tau[:, 1:2], thr)
    gq_ref[...] = gq
    gk_ref[...] = gk
    sq_ref[...] += jnp.sum(gq, axis=0, keepdims=True)
    sk_ref[...] += jnp.sum(gk, axis=0, keepdims=True)
    aux_ref[...] = jnp.reshape(_aux_of(sq_ref) + _aux_of(sk_ref), (1, 1))


def _single_kernel(x_ref, emb_ref, w_ref, b_ref, auxin_ref, g_ref, aux_ref,
                   s_ref):
    i = pl.program_id(0)

    @pl.when(i == 0)
    def _():
        s_ref[...] = jnp.zeros_like(s_ref)

    x = x_ref[...]
    scores = jax.lax.dot_general(
        x / 0.9, emb_ref[...], dimension_numbers=(((1,), (1,)), ((), ())),
        preferred_element_type=jnp.float32)
    tau = jnp.dot(x, w_ref[...], preferred_element_type=jnp.float32) + b_ref[...]
    g = _apply_gate(scores, tau, _score_threshold(scores, K))
    g_ref[...] = g
    s_ref[...] += jnp.sum(g, axis=0, keepdims=True)
    aux_ref[...] = auxin_ref[...] + jnp.reshape(_aux_of(s_ref), (1, 1))


def _common_specs():
    x_spec = pl.BlockSpec((TBLK, D), lambda i: (i, 0))
    emb_spec = pl.BlockSpec((N, D), lambda i: (0, 0))
    g_spec = pl.BlockSpec((TBLK, N), lambda i: (i, 0))
    aux_spec = pl.BlockSpec((1, 1), lambda i: (0, 0))
    return x_spec, emb_spec, g_spec, aux_spec


def kernel(x, qk_emb, v_emb, know_emb, W_tau_attn, b_tau_attn, W_tau_know,
           b_tau_know):
    x2 = x.reshape(S, D)
    x_spec, emb_spec, g_spec, aux_spec = _common_specs()
    w3_spec = pl.BlockSpec((D, 3), lambda i: (0, 0))
    w1_spec = pl.BlockSpec((D, 1), lambda i: (0, 0))
    b3 = b_tau_attn.reshape(1, 3)
    sum_scratch = pltpu.VMEM((1, N), jnp.float32)

    gq, gk, aux_qk = pl.pallas_call(
        _qk_kernel,
        grid=(GRID,),
        in_specs=[x_spec, emb_spec, w3_spec, pl.BlockSpec((1, 3), lambda i: (0, 0))],
        out_specs=[g_spec, g_spec, aux_spec],
        out_shape=[
            jax.ShapeDtypeStruct((S, N), jnp.float32),
            jax.ShapeDtypeStruct((S, N), jnp.float32),
            jax.ShapeDtypeStruct((1, 1), jnp.float32),
        ],
        scratch_shapes=[sum_scratch, sum_scratch],
    )(x2, qk_emb, W_tau_attn, b3)

    gv, aux_attn = pl.pallas_call(
        _single_kernel,
        grid=(GRID,),
        in_specs=[x_spec, emb_spec, w1_spec, aux_spec, aux_spec],
        out_specs=[g_spec, aux_spec],
        out_shape=[
            jax.ShapeDtypeStruct((S, N), jnp.float32),
            jax.ShapeDtypeStruct((1, 1), jnp.float32),
        ],
        scratch_shapes=[sum_scratch],
    )(x2, v_emb, W_tau_attn[:, 2:3], b_tau_attn[2].reshape(1, 1), aux_qk)

    gknow, aux_know = pl.pallas_call(
        _single_kernel,
        grid=(GRID,),
        in_specs=[x_spec, emb_spec, w1_spec, aux_spec, aux_spec],
        out_specs=[g_spec, aux_spec],
        out_shape=[
            jax.ShapeDtypeStruct((S, N), jnp.float32),
            jax.ShapeDtypeStruct((1, 1), jnp.float32),
        ],
        scratch_shapes=[sum_scratch],
    )(x2, know_emb, W_tau_know, b_tau_know.reshape(1, 1),
      jnp.zeros((1, 1), jnp.float32))

    shape = (B, S, N)
    return (gq.reshape(shape), gk.reshape(shape), gv.reshape(shape),
            aux_attn.reshape(()), gknow.reshape(shape), aux_know.reshape(()))
